# baseline (speedup 1.0000x reference)
import math

import jax
import jax.numpy as jnp
from jax.experimental import pallas as pl
from jax.experimental.pallas import tpu as pltpu

TOKENS = 8192
MODEL_DIM = 2048
MAX_E = 64
CLAMP_MAX = math.log(1.0 / 0.01)

NS = 2
BT = 512
SLICE = TOKENS // NS


def _gate_kernel(xa_ref, xb_ref, sim_ref, gates_ref, mask_ref, temp_ref,
                 out_ref, topk_ref, wn_ref, thr_ref):
    @pl.when(pl.program_id(0) == 0)
    def _prep():
        w = sim_ref[...]
        cn = jnp.sqrt(jnp.sum(w * w, axis=0, keepdims=True))
        wn_ref[...] = w / jnp.maximum(cn, 1e-12)
        scale = jnp.exp(jnp.minimum(temp_ref[0, 0], CLAMP_MAX))
        thr_ref[0:1, :] = jnp.where(mask_ref[...] > 0,
                                    gates_ref[...] * scale, jnp.inf)
        thr_ref[1:2, :] = jnp.full((1, MAX_E), scale)

    wn = wn_ref[...]
    thresh = thr_ref[0:1, :]
    scale = thr_ref[1, 0]

    def half(x):
        rn = jnp.sqrt(jnp.sum(x * x, axis=1, keepdims=True))
        xn = x / jnp.maximum(rn, 1e-12)
        s = jnp.dot(xn, wn, preferred_element_type=jnp.float32)
        out = jnp.where(s * scale > thresh, 1.0, 0.0)
        return out, jnp.sum(out, axis=1, keepdims=True).astype(jnp.int32)

    oa, ka = half(xa_ref[0])
    ob, kb = half(xb_ref[0])
    out_ref[0] = oa
    out_ref[1] = ob
    topk_ref[0] = ka
    topk_ref[1] = kb


def kernel(x, sim_matrix, gates, experts_mask, temperature):
    x3 = x.reshape(NS, SLICE, MODEL_DIM)
    gates2 = gates.reshape(1, MAX_E)
    mask2 = experts_mask.reshape(1, MAX_E)
    temp2 = temperature.reshape(1, 1)
    grid = (SLICE // BT,)
    logits_out, topk = pl.pallas_call(
        _gate_kernel,
        grid=grid,
        in_specs=[
            pl.BlockSpec((1, BT, MODEL_DIM), lambda i: (0, i, 0)),
            pl.BlockSpec((1, BT, MODEL_DIM), lambda i: (1, i, 0)),
            pl.BlockSpec((MODEL_DIM, MAX_E), lambda i: (0, 0)),
            pl.BlockSpec((1, MAX_E), lambda i: (0, 0)),
            pl.BlockSpec((1, MAX_E), lambda i: (0, 0)),
            pl.BlockSpec((1, 1), lambda i: (0, 0)),
        ],
        out_specs=[
            pl.BlockSpec((NS, BT, MAX_E), lambda i: (0, i, 0)),
            pl.BlockSpec((NS, BT, 1), lambda i: (0, i, 0)),
        ],
        out_shape=[
            jax.ShapeDtypeStruct((NS, SLICE, MAX_E), jnp.float32),
            jax.ShapeDtypeStruct((NS, SLICE, 1), jnp.int32),
        ],
        scratch_shapes=[
            pltpu.VMEM((MODEL_DIM, MAX_E), jnp.float32),
            pltpu.VMEM((2, MAX_E), jnp.float32),
        ],
        compiler_params=pltpu.CompilerParams(
            dimension_semantics=("arbitrary",),
        ),
    )(x3, x3, sim_matrix, gates2, mask2, temp2)
    return (logits_out.reshape(TOKENS, MAX_E), topk.reshape(TOKENS))


# speedup vs baseline: 1.0427x; 1.0427x over previous
import math

import jax
import jax.numpy as jnp
from jax.experimental import pallas as pl
from jax.experimental.pallas import tpu as pltpu

TOKENS = 8192
MODEL_DIM = 2048
MAX_E = 64
CLAMP_MAX = math.log(1.0 / 0.01)

BT = 2048


def _gate_kernel(x_ref, sim_ref, gates_ref, mask_ref, temp_ref,
                 out_ref, topk_ref, wn_ref, thr_ref):
    @pl.when(pl.program_id(0) == 0)
    def _prep():
        w = sim_ref[...]
        cn = jnp.sqrt(jnp.sum(w * w, axis=0, keepdims=True))
        wn_ref[...] = w / jnp.maximum(cn, 1e-12)
        scale = jnp.exp(jnp.minimum(temp_ref[0, 0], CLAMP_MAX))
        thr_ref[0:1, :] = jnp.where(mask_ref[...] > 0,
                                    gates_ref[...] * scale, jnp.inf)
        thr_ref[1:2, :] = jnp.full((1, MAX_E), scale)

    wn = wn_ref[...]
    thresh = thr_ref[0:1, :]
    scale = thr_ref[1, 0]

    x = x_ref[...]
    rn = jnp.sqrt(jnp.sum(x * x, axis=1, keepdims=True))
    xn = x / jnp.maximum(rn, 1e-12)
    s = jnp.dot(xn, wn, preferred_element_type=jnp.float32)
    out = jnp.where(s * scale > thresh, 1.0, 0.0)
    out_ref[...] = out
    topk_ref[...] = jnp.sum(out, axis=1, keepdims=True).astype(jnp.int32)


def kernel(x, sim_matrix, gates, experts_mask, temperature):
    gates2 = gates.reshape(1, MAX_E)
    mask2 = experts_mask.reshape(1, MAX_E)
    temp2 = temperature.reshape(1, 1)
    grid = (TOKENS // BT,)
    logits_out, topk = pl.pallas_call(
        _gate_kernel,
        grid=grid,
        in_specs=[
            pl.BlockSpec((BT, MODEL_DIM), lambda i: (i, 0)),
            pl.BlockSpec((MODEL_DIM, MAX_E), lambda i: (0, 0)),
            pl.BlockSpec((1, MAX_E), lambda i: (0, 0)),
            pl.BlockSpec((1, MAX_E), lambda i: (0, 0)),
            pl.BlockSpec((1, 1), lambda i: (0, 0)),
        ],
        out_specs=[
            pl.BlockSpec((BT, MAX_E), lambda i: (i, 0)),
            pl.BlockSpec((BT, 1), lambda i: (i, 0)),
        ],
        out_shape=[
            jax.ShapeDtypeStruct((TOKENS, MAX_E), jnp.float32),
            jax.ShapeDtypeStruct((TOKENS, 1), jnp.int32),
        ],
        scratch_shapes=[
            pltpu.VMEM((MODEL_DIM, MAX_E), jnp.float32),
            pltpu.VMEM((2, MAX_E), jnp.float32),
        ],
        compiler_params=pltpu.CompilerParams(
            dimension_semantics=("arbitrary",),
        ),
    )(x, sim_matrix, gates2, mask2, temp2)
    return (logits_out, topk.reshape(TOKENS))
